# baseline (speedup 1.0000x reference)
import functools

import jax
import jax.numpy as jnp
from jax import lax
from jax.experimental import pallas as pl
from jax.experimental.pallas import tpu as pltpu
from jax.experimental.pallas import tpu_sc as plsc

_ROWS, _COLS = 16384, 4096
_NC, _NS = 2, 16
_NW = _NC * _NS
_RPW = _ROWS // _NW
_CH = 8
_NBUF = 2
_NCHUNKS = _RPW // _CH
_NGROUPS = _NCHUNKS // _NBUF


def _sc_body(x_hbm, o_hbm, buf, in_sems, out_sems):
    wid = lax.axis_index("s") * _NC + lax.axis_index("c")
    base = wid * _RPW

    def start_in(b, ci):
        pltpu.make_async_copy(
            x_hbm.at[pl.ds(base + ci * _CH, _CH), :], buf.at[b], in_sems.at[b]
        ).start()

    def wait_in(b):
        pltpu.make_async_copy(
            x_hbm.at[pl.ds(0, _CH), :], buf.at[b], in_sems.at[b]
        ).wait()

    def start_out(b, ci):
        pltpu.make_async_copy(
            buf.at[b], o_hbm.at[pl.ds(base + ci * _CH, _CH), :], out_sems.at[b]
        ).start()

    def wait_out(b):
        pltpu.make_async_copy(
            buf.at[b], o_hbm.at[pl.ds(0, _CH), :], out_sems.at[b]
        ).wait()

    for b in range(_NBUF):
        start_in(b, b)

    def group(g, _):
        for b in range(_NBUF):
            wait_in(b)
            if b == 0:
                @pl.when(jnp.logical_and(wid == 0, g == 0))
                def _patch():
                    i16 = lax.iota(jnp.int32, 16)
                    v0 = buf[0, 0, pl.ds(0, 16)]
                    v0 = jnp.where(i16 == 0, 10.0, jnp.where(i16 == 2, 30.0, v0))
                    buf[0, 0, pl.ds(0, 16)] = v0
                    v1 = buf[0, 1, pl.ds(0, 16)]
                    v1 = jnp.where(i16 == 0, 40.0, jnp.where(i16 == 1, 20.0, v1))
                    buf[0, 1, pl.ds(0, 16)] = v1

            start_out(b, g * _NBUF + b)
        for b in range(_NBUF):
            @pl.when(g + 1 < _NGROUPS)
            def _prefetch():
                wait_out(b)
                start_in(b, (g + 1) * _NBUF + b)
        return 0

    lax.fori_loop(0, _NGROUPS, group, 0)
    for b in range(_NBUF):
        wait_out(b)


def kernel(x):
    mesh = plsc.VectorSubcoreMesh(core_axis_name="c", subcore_axis_name="s")
    run = functools.partial(
        pl.kernel,
        mesh=mesh,
        out_type=jax.ShapeDtypeStruct((_ROWS, _COLS), jnp.float32),
        scratch_types=[
            pltpu.VMEM((_NBUF, _CH, _COLS), jnp.float32),
            pltpu.SemaphoreType.DMA((_NBUF,)),
            pltpu.SemaphoreType.DMA((_NBUF,)),
        ],
    )(_sc_body)
    return run(x)


# speedup vs baseline: 1.1470x; 1.1470x over previous
import functools

import jax
import jax.numpy as jnp
from jax import lax
from jax.experimental import pallas as pl
from jax.experimental.pallas import tpu as pltpu
from jax.experimental.pallas import tpu_sc as plsc

_ROWS, _COLS = 16384, 4096
_BLOCK = 512
_NC = 2
_PR = 8


def _tc_copy_body(x_ref, o_ref):
    o_ref[...] = x_ref[...]


def _tc_copy(x):
    return pl.pallas_call(
        _tc_copy_body,
        grid=(_ROWS // _BLOCK,),
        in_specs=[pl.BlockSpec((_BLOCK, _COLS), lambda i: (i, 0))],
        out_specs=pl.BlockSpec((_BLOCK, _COLS), lambda i: (i, 0)),
        out_shape=jax.ShapeDtypeStruct((_ROWS, _COLS), jnp.float32),
    )(x)


def _sc_scatter_body(x_hbm, o_hbm, buf, sem):
    wid = lax.axis_index("s") * _NC + lax.axis_index("c")

    @pl.when(wid == 0)
    def _work():
        pltpu.make_async_copy(x_hbm.at[pl.ds(0, _PR), :], buf, sem).start()
        pltpu.make_async_copy(x_hbm.at[pl.ds(0, _PR), :], buf, sem).wait()
        i16 = lax.iota(jnp.int32, 16)
        v0 = buf[0, pl.ds(0, 16)]
        v0 = jnp.where(i16 == 0, 10.0, jnp.where(i16 == 2, 30.0, v0))
        buf[0, pl.ds(0, 16)] = v0
        v1 = buf[1, pl.ds(0, 16)]
        v1 = jnp.where(i16 == 0, 40.0, jnp.where(i16 == 1, 20.0, v1))
        buf[1, pl.ds(0, 16)] = v1
        pltpu.make_async_copy(buf, o_hbm, sem).start()
        pltpu.make_async_copy(buf, o_hbm, sem).wait()


def _sc_scatter_rows(x):
    mesh = plsc.VectorSubcoreMesh(core_axis_name="c", subcore_axis_name="s")
    run = functools.partial(
        pl.kernel,
        mesh=mesh,
        out_type=jax.ShapeDtypeStruct((_PR, _COLS), jnp.float32),
        scratch_types=[
            pltpu.VMEM((_PR, _COLS), jnp.float32),
            pltpu.SemaphoreType.DMA,
        ],
    )(_sc_scatter_body)
    return run(x)


def kernel(x):
    big = _tc_copy(x)
    patched = _sc_scatter_rows(x)
    return lax.dynamic_update_slice(big, patched[0:2], (0, 0))


# speedup vs baseline: 1.1482x; 1.0011x over previous
import functools

import jax
import jax.numpy as jnp
from jax import lax
from jax.experimental import pallas as pl
from jax.experimental.pallas import tpu as pltpu
from jax.experimental.pallas import tpu_sc as plsc

_ROWS, _COLS = 16384, 4096
_BLOCK = 512
_NC = 2
_PR = 8


def _tc_copy_body(x_ref, o_ref):
    o_ref[...] = x_ref[...]


def _tc_copy(x):
    return pl.pallas_call(
        _tc_copy_body,
        grid=(_ROWS // _BLOCK,),
        in_specs=[pl.BlockSpec((_BLOCK, _COLS), lambda i: (i, 0))],
        out_specs=pl.BlockSpec((_BLOCK, _COLS), lambda i: (i, 0)),
        out_shape=jax.ShapeDtypeStruct((_ROWS, _COLS), jnp.float32),
    )(x)


def _sc_scatter_body(x_hbm, o_hbm, buf, sem):
    wid = lax.axis_index("s") * _NC + lax.axis_index("c")

    @pl.when(wid == 0)
    def _work():
        pltpu.make_async_copy(x_hbm.at[pl.ds(0, _PR), :], buf, sem).start()
        pltpu.make_async_copy(x_hbm.at[pl.ds(0, _PR), :], buf, sem).wait()
        i16 = lax.iota(jnp.int32, 16)
        v0 = buf[0, pl.ds(0, 16)]
        v0 = jnp.where(i16 == 0, 10.0, jnp.where(i16 == 2, 30.0, v0))
        buf[0, pl.ds(0, 16)] = v0
        v1 = buf[1, pl.ds(0, 16)]
        v1 = jnp.where(i16 == 0, 40.0, jnp.where(i16 == 1, 20.0, v1))
        buf[1, pl.ds(0, 16)] = v1
        pltpu.make_async_copy(buf, o_hbm, sem).start()
        pltpu.make_async_copy(buf, o_hbm, sem).wait()


def _sc_scatter_rows(x):
    mesh = plsc.VectorSubcoreMesh(core_axis_name="c", subcore_axis_name="s")
    run = functools.partial(
        pl.kernel,
        mesh=mesh,
        out_type=jax.ShapeDtypeStruct((_PR, _COLS), jnp.float32),
        scratch_types=[
            pltpu.VMEM((_PR, _COLS), jnp.float32),
            pltpu.SemaphoreType.DMA,
        ],
    )(_sc_scatter_body)
    return run(x)


def kernel(x):
    patched = _sc_scatter_rows(x)
    big = _tc_copy(x)
    return lax.dynamic_update_slice(big, patched[0:2], (0, 0))


# speedup vs baseline: 1.2668x; 1.1033x over previous
import jax
import jax.numpy as jnp
from jax.experimental import pallas as pl

_ROWS, _COLS = 16384, 4096
_BLOCK = 512


def _copy_patch_kernel(x_ref, o_ref):
    o_ref[...] = x_ref[...]

    @pl.when(pl.program_id(0) == 0)
    def _patch():
        tile = o_ref[0:8, 0:128]
        r = jax.lax.broadcasted_iota(jnp.int32, (8, 128), 0)
        c = jax.lax.broadcasted_iota(jnp.int32, (8, 128), 1)
        tile = jnp.where((r == 0) & (c == 0), 10.0, tile)
        tile = jnp.where((r == 0) & (c == 2), 30.0, tile)
        tile = jnp.where((r == 1) & (c == 0), 40.0, tile)
        tile = jnp.where((r == 1) & (c == 1), 20.0, tile)
        o_ref[0:8, 0:128] = tile


def kernel(x):
    return pl.pallas_call(
        _copy_patch_kernel,
        grid=(_ROWS // _BLOCK,),
        in_specs=[pl.BlockSpec((_BLOCK, _COLS), lambda i: (i, 0))],
        out_specs=pl.BlockSpec((_BLOCK, _COLS), lambda i: (i, 0)),
        out_shape=jax.ShapeDtypeStruct((_ROWS, _COLS), jnp.float32),
    )(x)
